# baseline (speedup 1.0000x reference)
import functools
import math

import jax
import jax.numpy as jnp
from jax.experimental import pallas as pl
from jax.experimental.pallas import tpu as pltpu

_PAR = pltpu.CompilerParams(dimension_semantics=("parallel",))

B = 8
N = 2048
GFD = 1024
AFD = 64
NS1 = int(math.ceil(0.2 * N))
NS2 = int(math.ceil(0.25 * NS1))
R1 = 0.2
R2 = 0.4
KNBR = 64

_INTERPRET = False
_HI = jax.lax.Precision.HIGHEST
INF = float('inf')


def _dot(a, b):
    return jnp.dot(a, b, preferred_element_type=jnp.float32)


def _gdot(a, b):
    return jnp.dot(a, b, precision=_HI, preferred_element_type=jnp.float32)


def _mlp(x, layers):
    n = len(layers)
    for i, (W, b) in enumerate(layers):
        x = _dot(x, W) + b
        if i < n - 1:
            x = jax.nn.relu(x)
    return x


def _argmax_low(v):
    m = jnp.max(v, axis=1, keepdims=True)
    n = v.shape[1]
    ii = jax.lax.broadcasted_iota(jnp.int32, v.shape, 1)
    idx = jnp.min(jnp.where(v == m, ii, n), axis=1, keepdims=True)
    return m, idx


def _fps_kernel(posT_ref, sel_ref, *, ns):
    px = posT_ref[:, 0, :]
    py = posT_ref[:, 1, :]
    pz = posT_ref[:, 2, :]
    n = px.shape[1]
    d0 = ((px - px[:, 0:1]) ** 2 + (py - py[:, 0:1]) ** 2) + (pz - pz[:, 0:1]) ** 2
    jj = jax.lax.broadcasted_iota(jnp.int32, (B, n), 1)
    cols = jax.lax.broadcasted_iota(jnp.int32, (B, ns), 1)
    sel0 = jnp.zeros((B, ns), jnp.float32)

    def body(i, carry):
        sel, d = carry
        m = jnp.max(d, axis=1, keepdims=True)
        jmin = jnp.min(jnp.where(d == m, jj, n), axis=1, keepdims=True)
        oh = jj == jmin
        sxi = jnp.sum(jnp.where(oh, px, 0.0), axis=1, keepdims=True)
        syi = jnp.sum(jnp.where(oh, py, 0.0), axis=1, keepdims=True)
        szi = jnp.sum(jnp.where(oh, pz, 0.0), axis=1, keepdims=True)
        nd = ((px - sxi) ** 2 + (py - syi) ** 2) + (pz - szi) ** 2
        sel = sel + (cols == i).astype(jnp.float32) * jmin.astype(jnp.float32)
        return sel, jnp.minimum(d, nd)

    sel, _ = jax.lax.fori_loop(1, ns, body, (sel0, d0))
    sel_ref[...] = sel


def _fps_call(posT, ns):
    sel = pl.pallas_call(
        functools.partial(_fps_kernel, ns=ns),
        in_specs=[_full_spec(posT.shape)],
        out_specs=_full_spec((B, ns)),
        out_shape=jax.ShapeDtypeStruct((B, ns), jnp.float32),
        interpret=_INTERPRET,
    )(posT)
    return sel.reshape(B, ns, 1)


def _sa_kernel(posT_ref, x_ref, sel_ref,
               w1x_ref, w1p_ref, b1_ref, w2_ref, b2_ref, w3_ref, b3_ref,
               out_ref, poss_ref, *, ns, r2, nk, x_is_pos):
    posT = posT_ref[0]
    x = x_ref[0]
    n = posT.shape[1]
    px = posT[0:1, :]
    py = posT[1:2, :]
    pz = posT[2:3, :]

    selcol = sel_ref[0].astype(jnp.int32)
    jj = jax.lax.broadcasted_iota(jnp.int32, (ns, n), 1)
    ohs = jj == selcol
    psx = jnp.sum(jnp.where(ohs, px, 0.0), axis=1, keepdims=True)
    psy = jnp.sum(jnp.where(ohs, py, 0.0), axis=1, keepdims=True)
    psz = jnp.sum(jnp.where(ohs, pz, 0.0), axis=1, keepdims=True)
    psc = jnp.concatenate([psx, psy, psz], axis=1)

    d2 = (((psc[:, 0:1] - px) ** 2 + (psc[:, 1:2] - py) ** 2)
          + (psc[:, 2:3] - pz) ** 2)

    out0 = jnp.full((ns, w3_ref.shape[1]), -INF, jnp.float32)
    w1 = jnp.concatenate([w1x_ref[...], w1p_ref[...]], axis=0)

    def nb_body(_, carry):
        d2c, out = carry
        m = jnp.min(d2c, axis=1, keepdims=True)
        jmin = jnp.min(jnp.where(d2c == m, jj, n), axis=1, keepdims=True)
        ohb = jj == jmin
        d2c = jnp.where(ohb, INF, d2c)
        gx = jnp.sum(jnp.where(ohb, px, 0.0), axis=1, keepdims=True)
        gy = jnp.sum(jnp.where(ohb, py, 0.0), axis=1, keepdims=True)
        gz = jnp.sum(jnp.where(ohb, pz, 0.0), axis=1, keepdims=True)
        gathered = jnp.concatenate([gx, gy, gz], axis=1)
        if x_is_pos:
            xsel = gathered
        else:
            xsel = _gdot(ohb.astype(jnp.float32), x)
        rel = gathered - psc
        msg = jnp.concatenate([xsel, rel], axis=1)
        h = jax.nn.relu(_dot(msg, w1) + b1_ref[...])
        h = jax.nn.relu(_dot(h, w2_ref[...]) + b2_ref[...])
        h = _dot(h, w3_ref[...]) + b3_ref[...]
        h = jnp.where(m <= r2, h, -INF)
        return d2c, jnp.maximum(out, h)

    _, out = jax.lax.fori_loop(0, nk, nb_body, (d2, out0))
    out_ref[0] = out
    poss_ref[0] = psc


def _sa_call(posT, x, sel, layers, ns, r2, x_is_pos):
    (W1, b1), (W2, b2), (W3, b3) = [(w, b.reshape(1, -1)) for w, b in layers]
    c = x.shape[-1]
    w1x, w1p = W1[:c], W1[c:]
    n = posT.shape[2]
    h3 = W3.shape[1]
    spec3 = lambda s: pl.BlockSpec((1,) + s[1:], lambda b: (b, 0, 0))
    wspec = lambda s: pl.BlockSpec(s, lambda b: (0, 0))
    return pl.pallas_call(
        functools.partial(_sa_kernel, ns=ns, r2=r2, nk=min(KNBR, n),
                          x_is_pos=x_is_pos),
        grid=(B,),
        in_specs=[spec3(posT.shape), spec3(x.shape), spec3(sel.shape),
                  wspec(w1x.shape), wspec(w1p.shape), wspec(b1.shape),
                  wspec(W2.shape), wspec(b2.shape),
                  wspec(W3.shape), wspec(b3.shape)],
        out_specs=[pl.BlockSpec((1, ns, h3), lambda b: (b, 0, 0)),
                   pl.BlockSpec((1, ns, 3), lambda b: (b, 0, 0))],
        out_shape=[jax.ShapeDtypeStruct((B, ns, h3), jnp.float32),
                   jax.ShapeDtypeStruct((B, ns, 3), jnp.float32)],
        compiler_params=_PAR,
        interpret=_INTERPRET,
    )(posT, x, sel, w1x, w1p, b1, W2, b2, W3, b3)


def _knn3_interp(d2, xsrc, nsrc):
    jj = jax.lax.broadcasted_iota(jnp.int32, d2.shape, 1)
    num = jnp.zeros((d2.shape[0], xsrc.shape[1]), jnp.float32)
    den = jnp.zeros((d2.shape[0], 1), jnp.float32)
    for _ in range(3):
        m = jnp.min(d2, axis=1, keepdims=True)
        jmin = jnp.min(jnp.where(d2 == m, jj, nsrc), axis=1, keepdims=True)
        oh = (jj == jmin).astype(jnp.float32)
        d2 = jnp.where(jj == jmin, INF, d2)
        w = 1.0 / jnp.maximum(m, 1e-16)
        num = num + _gdot(oh, xsrc) * w
        den = den + w
    return num / den


def _mid_kernel(x2_ref, pos2_ref, pos2T_ref, x1_ref, pos1_ref,
                s31_ref, s31b, s32_ref, s32b, s33_ref, s33b,
                f31_ref, f31b, f32_ref, f32b,
                f21_ref, f21b, f22_ref, f22b,
                gmax_ref, xf2_ref):
    x2 = x2_ref[0]
    pos2 = pos2_ref[0]
    pos2T = pos2T_ref[0]
    x1 = x1_ref[0]
    pos1 = pos1_ref[0]

    g = _mlp(jnp.concatenate([x2, pos2], axis=1),
             [(s31_ref[...], s31b[...]), (s32_ref[...], s32b[...]),
              (s33_ref[...], s33b[...])])
    gmax = jnp.max(g, axis=0, keepdims=True)
    gmax_ref[0] = gmax

    d2z = (pos2[:, 0:1] ** 2 + pos2[:, 1:2] ** 2) + pos2[:, 2:3] ** 2
    w = 1.0 / jnp.maximum(d2z, 1e-16)
    y3 = (gmax * w) / w
    x3 = _mlp(jnp.concatenate([y3, x2], axis=1),
              [(f31_ref[...], f31b[...]), (f32_ref[...], f32b[...])])

    d2 = (((pos1[:, 0:1] - pos2T[0:1, :]) ** 2
           + (pos1[:, 1:2] - pos2T[1:2, :]) ** 2)
          + (pos1[:, 2:3] - pos2T[2:3, :]) ** 2)
    y2 = _knn3_interp(d2, x3, pos2T.shape[1])
    xf2_ref[0] = _mlp(jnp.concatenate([y2, x1], axis=1),
                      [(f21_ref[...], f21b[...]), (f22_ref[...], f22b[...])])


def _ape_kernel(posT_ref, loc_ref, a1_ref, a1b, a2_ref, a2b, af_ref, ap_ref):
    loc = loc_ref[...].astype(jnp.int32)
    ii = jax.lax.broadcasted_iota(jnp.int32, (B, N), 1)
    oh = ii == loc
    ax = jnp.sum(jnp.where(oh, posT_ref[:, 0, :], 0.0), axis=1, keepdims=True)
    ay = jnp.sum(jnp.where(oh, posT_ref[:, 1, :], 0.0), axis=1, keepdims=True)
    az = jnp.sum(jnp.where(oh, posT_ref[:, 2, :], 0.0), axis=1, keepdims=True)
    ap = jnp.concatenate([ax, ay, az], axis=1)
    ap_ref[...] = ap
    af_ref[:, 0, :] = _mlp(ap, [(a1_ref[...], a1b[...]), (a2_ref[...], a2b[...])])


def _big_kernel(xf2_ref, pos1T_ref, poscol_ref, af_ref,
                f11_ref, f11b, f12_ref, f12b, f13_ref, f13b,
                g1_ref, g1b, g2_ref, g2b, g3_ref, g3b,
                prob_ref):
    xf2 = xf2_ref[0]
    pos1T = pos1T_ref[0]
    poscol = poscol_ref[0]
    af = af_ref[0]

    d2 = (((poscol[:, 0:1] - pos1T[0:1, :]) ** 2
           + (poscol[:, 1:2] - pos1T[1:2, :]) ** 2)
          + (poscol[:, 2:3] - pos1T[2:3, :]) ** 2)
    y1 = _knn3_interp(d2, xf2, pos1T.shape[1])
    xf1 = _mlp(jnp.concatenate([y1, poscol], axis=1),
               [(f11_ref[...], f11b[...]), (f12_ref[...], f12b[...]),
                (f13_ref[...], f13b[...])])
    feat = jnp.concatenate([xf1, jnp.broadcast_to(af, (xf1.shape[0], af.shape[1]))],
                           axis=1)
    h = _mlp(feat, [(g1_ref[...], g1b[...]), (g2_ref[...], g2b[...]),
                    (g3_ref[...], g3b[...])])
    prob_ref[0] = jax.nn.sigmoid(h)


def _head_kernel(prob_ref, posT_ref, ap_ref, gmax_ref,
                 t1_ref, t1b, t2_ref, t2b,
                 p1_ref, p1b, p2_ref, p2b, p3_ref, p3b,
                 gr_ref, len_ref, dot_ref):
    prob = prob_ref[...]
    ii = jax.lax.broadcasted_iota(jnp.int32, (B, N), 1)
    gps = []
    for _ in range(2):
        m = jnp.max(prob, axis=1, keepdims=True)
        jmin = jnp.min(jnp.where(prob == m, ii, N), axis=1, keepdims=True)
        oh = ii == jmin
        prob = jnp.where(oh, -INF, prob)
        gx = jnp.sum(jnp.where(oh, posT_ref[:, 0, :], 0.0), axis=1, keepdims=True)
        gy = jnp.sum(jnp.where(oh, posT_ref[:, 1, :], 0.0), axis=1, keepdims=True)
        gz = jnp.sum(jnp.where(oh, posT_ref[:, 2, :], 0.0), axis=1, keepdims=True)
        gps.append(jnp.concatenate([gx, gy, gz], axis=1))
    p1, p2 = gps
    ap = ap_ref[...]
    cond = jnp.concatenate([ap, p1, p2], axis=1)
    tf = _mlp(cond, [(t1_ref[...], t1b[...]), (t2_ref[...], t2b[...])])
    trans = _mlp(jnp.concatenate([gmax_ref[...], tf], axis=1),
                 [(p1_ref[...], p1b[...]), (p2_ref[...], p2b[...]),
                  (p3_ref[...], p3b[...])])
    z = p2 - p1
    zn = jnp.sqrt(jnp.sum(z * z, axis=1, keepdims=True))
    z = z / (zn + 1e-12)
    mid = (p1 + p2) / 2.0
    xv = trans - mid
    length = jnp.sqrt(jnp.sum(xv * xv, axis=1, keepdims=True))
    xa = xv / (length + 1e-12)
    dzx = jnp.sum(z * xa, axis=1, keepdims=True)
    ya = jnp.concatenate(
        [z[:, 1:2] * xa[:, 2:3] - z[:, 2:3] * xa[:, 1:2],
         z[:, 2:3] * xa[:, 0:1] - z[:, 0:1] * xa[:, 2:3],
         z[:, 0:1] * xa[:, 1:2] - z[:, 1:2] * xa[:, 0:1]], axis=1)
    cols = []
    for i in range(3):
        cols += [xa[:, i:i + 1], ya[:, i:i + 1], z[:, i:i + 1], mid[:, i:i + 1]]
    gr_ref[...] = jnp.concatenate(cols, axis=1)
    len_ref[...] = length
    dot_ref[...] = dzx


def _full_spec(shape):
    return pl.BlockSpec(shape, lambda: tuple(0 for _ in shape))


def _mid_call(x2, pos2, pos2T, x1, pos1, params):
    (s31, s31b), (s32, s32b), (s33, s33b) = [(w, b.reshape(1, -1)) for w, b in params['sa3']]
    (f31, f31b), (f32, f32b) = [(w, b.reshape(1, -1)) for w, b in params['fp3']]
    (f21, f21b), (f22, f22b) = [(w, b.reshape(1, -1)) for w, b in params['fp2']]
    spec3 = lambda s: pl.BlockSpec((1,) + s[1:], lambda b: (b, 0, 0))
    wspec = lambda a: pl.BlockSpec(a.shape, lambda b: (0, 0))
    gmax3, xf2 = pl.pallas_call(
        _mid_kernel,
        grid=(B,),
        in_specs=[spec3(x2.shape), spec3(pos2.shape), spec3(pos2T.shape),
                  spec3(x1.shape), spec3(pos1.shape)]
                 + [wspec(a) for a in (s31, s31b, s32, s32b, s33, s33b,
                                       f31, f31b, f32, f32b,
                                       f21, f21b, f22, f22b)],
        out_specs=[pl.BlockSpec((1, 1, GFD), lambda b: (b, 0, 0)),
                   pl.BlockSpec((1, NS1, 128), lambda b: (b, 0, 0))],
        out_shape=[jax.ShapeDtypeStruct((B, 1, GFD), jnp.float32),
                   jax.ShapeDtypeStruct((B, NS1, 128), jnp.float32)],
        compiler_params=_PAR,
        interpret=_INTERPRET,
    )(x2, pos2, pos2T, x1, pos1,
      s31, s31b, s32, s32b, s33, s33b, f31, f31b, f32, f32b,
      f21, f21b, f22, f22b)
    return gmax3, xf2


def _ape_call(posT, aploc, params):
    (a1, a1b), (a2, a2b) = [(w, b.reshape(1, -1)) for w, b in params['ape']]
    af3, ap = pl.pallas_call(
        _ape_kernel,
        in_specs=[_full_spec(posT.shape), _full_spec((B, 1)),
                  _full_spec(a1.shape), _full_spec(a1b.shape),
                  _full_spec(a2.shape), _full_spec(a2b.shape)],
        out_specs=[_full_spec((B, 1, AFD)), _full_spec((B, 3))],
        out_shape=[jax.ShapeDtypeStruct((B, 1, AFD), jnp.float32),
                   jax.ShapeDtypeStruct((B, 3), jnp.float32)],
        interpret=_INTERPRET,
    )(posT, aploc, a1, a1b, a2, a2b)
    return af3, ap


def _big_call(xf2, pos1T, posr, af3, params):
    spec3 = lambda s: pl.BlockSpec((1,) + s[1:], lambda b: (b, 0, 0))
    wspec = lambda a: pl.BlockSpec(a.shape, lambda b: (0, 0))
    (f11, f11b), (f12, f12b), (f13, f13b) = [(w, b.reshape(1, -1)) for w, b in params['fp1']]
    (g1, g1b), (g2, g2b), (g3, g3b) = [(w, b.reshape(1, -1)) for w, b in params['gmlp']]
    prob3 = pl.pallas_call(
        _big_kernel,
        grid=(B,),
        in_specs=[spec3(xf2.shape), spec3(pos1T.shape), spec3(posr.shape),
                  spec3(af3.shape)]
                 + [wspec(a) for a in (f11, f11b, f12, f12b, f13, f13b,
                                       g1, g1b, g2, g2b, g3, g3b)],
        out_specs=pl.BlockSpec((1, N, 1), lambda b: (b, 0, 0)),
        out_shape=jax.ShapeDtypeStruct((B, N, 1), jnp.float32),
        compiler_params=_PAR,
        interpret=_INTERPRET,
    )(xf2, pos1T, posr, af3, f11, f11b, f12, f12b, f13, f13b,
      g1, g1b, g2, g2b, g3, g3b)
    return prob3


def _head_call(prob, posT, ap, gmax, params):
    (t1, t1b), (t2, t2b) = [(w, b.reshape(1, -1)) for w, b in params['tenc']]
    (q1, q1b), (q2, q2b), (q3, q3b) = [(w, b.reshape(1, -1)) for w, b in params['tpred']]
    gr12, length, dzx = pl.pallas_call(
        _head_kernel,
        in_specs=[_full_spec((B, N)), _full_spec(posT.shape), _full_spec((B, 3)),
                  _full_spec((B, GFD))]
                 + [_full_spec(a.shape) for a in (t1, t1b, t2, t2b,
                                                  q1, q1b, q2, q2b, q3, q3b)],
        out_specs=[_full_spec((B, 12)), _full_spec((B, 1)), _full_spec((B, 1))],
        out_shape=[jax.ShapeDtypeStruct((B, 12), jnp.float32),
                   jax.ShapeDtypeStruct((B, 1), jnp.float32),
                   jax.ShapeDtypeStruct((B, 1), jnp.float32)],
        interpret=_INTERPRET,
    )(prob, posT, ap, gmax, t1, t1b, t2, t2b, q1, q1b, q2, q2b, q3, q3b)
    return gr12, length, dzx


def kernel(pos, batch, approach_point_idx, params):
    del batch
    posr = pos.reshape(B, N, 3)
    posT = jnp.transpose(posr, (0, 2, 1))

    sel1 = _fps_call(posT, NS1)
    x1, pos1 = _sa_call(posT, posr, sel1, params['sa1'], NS1, R1 * R1, True)
    pos1T = jnp.transpose(pos1, (0, 2, 1))
    sel2 = _fps_call(pos1T, NS2)
    x2, pos2 = _sa_call(pos1T, x1, sel2, params['sa2'], NS2, R2 * R2, False)
    pos2T = jnp.transpose(pos2, (0, 2, 1))

    gmax3, xf2 = _mid_call(x2, pos2, pos2T, x1, pos1, params)

    aploc = (approach_point_idx - jnp.arange(B) * N).astype(jnp.float32).reshape(B, 1)
    af3, ap = _ape_call(posT, aploc, params)

    prob3 = _big_call(xf2, pos1T, posr, af3, params)
    prob = prob3.reshape(B, N)
    gmax = gmax3.reshape(B, GFD)

    gr12, length, dzx = _head_call(prob, posT, ap, gmax, params)
    return gr12.reshape(B, 3, 4), length.reshape(B), dzx.reshape(B)


# speedup vs baseline: 1.0008x; 1.0008x over previous
import functools
import math

import jax
import jax.numpy as jnp
from jax.experimental import pallas as pl
from jax.experimental.pallas import tpu as pltpu

_PAR = pltpu.CompilerParams(dimension_semantics=("parallel",))

B = 8
N = 2048
GFD = 1024
AFD = 64
NS1 = int(math.ceil(0.2 * N))
NS2 = int(math.ceil(0.25 * NS1))
R1 = 0.2
R2 = 0.4
KNBR = 64

_HI = jax.lax.Precision.HIGHEST
INF = float('inf')


def _dot(a, b):
    return jnp.dot(a, b, preferred_element_type=jnp.float32)


def _gdot(a, b):
    return jnp.dot(a, b, precision=_HI, preferred_element_type=jnp.float32)


def _mlp(x, layers):
    n = len(layers)
    for i, (W, b) in enumerate(layers):
        x = _dot(x, W) + b
        if i < n - 1:
            x = jax.nn.relu(x)
    return x


def _argmax_low(v):
    m = jnp.max(v, axis=1, keepdims=True)
    n = v.shape[1]
    ii = jax.lax.broadcasted_iota(jnp.int32, v.shape, 1)
    idx = jnp.min(jnp.where(v == m, ii, n), axis=1, keepdims=True)
    return m, idx


def _fps_kernel(posT_ref, sel_ref, *, ns):
    px = posT_ref[:, 0, :]
    py = posT_ref[:, 1, :]
    pz = posT_ref[:, 2, :]
    n = px.shape[1]
    d0 = ((px - px[:, 0:1]) ** 2 + (py - py[:, 0:1]) ** 2) + (pz - pz[:, 0:1]) ** 2
    jj = jax.lax.broadcasted_iota(jnp.int32, (B, n), 1)
    cols = jax.lax.broadcasted_iota(jnp.int32, (B, ns), 1)
    sel0 = jnp.zeros((B, ns), jnp.float32)

    def body(i, carry):
        sel, d = carry
        m = jnp.max(d, axis=1, keepdims=True)
        jmin = jnp.min(jnp.where(d == m, jj, n), axis=1, keepdims=True)
        oh = jj == jmin
        sxi = jnp.sum(jnp.where(oh, px, 0.0), axis=1, keepdims=True)
        syi = jnp.sum(jnp.where(oh, py, 0.0), axis=1, keepdims=True)
        szi = jnp.sum(jnp.where(oh, pz, 0.0), axis=1, keepdims=True)
        nd = ((px - sxi) ** 2 + (py - syi) ** 2) + (pz - szi) ** 2
        sel = sel + (cols == i).astype(jnp.float32) * jmin.astype(jnp.float32)
        return sel, jnp.minimum(d, nd)

    sel, _ = jax.lax.fori_loop(1, ns, body, (sel0, d0))
    sel_ref[...] = sel


def _fps_call(posT, ns):
    sel = pl.pallas_call(
        functools.partial(_fps_kernel, ns=ns),
        in_specs=[_full_spec(posT.shape)],
        out_specs=_full_spec((B, ns)),
        out_shape=jax.ShapeDtypeStruct((B, ns), jnp.float32),
    )(posT)
    return sel.reshape(B, ns, 1)


def _sa_kernel(posT_ref, x_ref, sel_ref,
               w1x_ref, w1p_ref, b1_ref, w2_ref, b2_ref, w3_ref, b3_ref,
               out_ref, poss_ref, *, ns, r2, nk, x_is_pos):
    posT = posT_ref[0]
    x = x_ref[0]
    n = posT.shape[1]
    px = posT[0:1, :]
    py = posT[1:2, :]
    pz = posT[2:3, :]

    selcol = sel_ref[0].astype(jnp.int32)
    jj = jax.lax.broadcasted_iota(jnp.int32, (ns, n), 1)
    ohs = jj == selcol
    psx = jnp.sum(jnp.where(ohs, px, 0.0), axis=1, keepdims=True)
    psy = jnp.sum(jnp.where(ohs, py, 0.0), axis=1, keepdims=True)
    psz = jnp.sum(jnp.where(ohs, pz, 0.0), axis=1, keepdims=True)
    psc = jnp.concatenate([psx, psy, psz], axis=1)

    d2 = (((psc[:, 0:1] - px) ** 2 + (psc[:, 1:2] - py) ** 2)
          + (psc[:, 2:3] - pz) ** 2)

    out0 = jnp.full((ns, w3_ref.shape[1]), -INF, jnp.float32)
    w1 = jnp.concatenate([w1x_ref[...], w1p_ref[...]], axis=0)

    def nb_body(_, carry):
        d2c, out = carry
        m = jnp.min(d2c, axis=1, keepdims=True)
        jmin = jnp.min(jnp.where(d2c == m, jj, n), axis=1, keepdims=True)
        ohb = jj == jmin
        d2c = jnp.where(ohb, INF, d2c)
        gx = jnp.sum(jnp.where(ohb, px, 0.0), axis=1, keepdims=True)
        gy = jnp.sum(jnp.where(ohb, py, 0.0), axis=1, keepdims=True)
        gz = jnp.sum(jnp.where(ohb, pz, 0.0), axis=1, keepdims=True)
        gathered = jnp.concatenate([gx, gy, gz], axis=1)
        if x_is_pos:
            xsel = gathered
        else:
            xsel = _gdot(ohb.astype(jnp.float32), x)
        rel = gathered - psc
        msg = jnp.concatenate([xsel, rel], axis=1)
        h = jax.nn.relu(_dot(msg, w1) + b1_ref[...])
        h = jax.nn.relu(_dot(h, w2_ref[...]) + b2_ref[...])
        h = _dot(h, w3_ref[...]) + b3_ref[...]
        h = jnp.where(m <= r2, h, -INF)
        return d2c, jnp.maximum(out, h)

    _, out = jax.lax.fori_loop(0, nk, nb_body, (d2, out0))
    out_ref[0] = out
    poss_ref[0] = psc


def _sa_call(posT, x, sel, layers, ns, r2, x_is_pos):
    (W1, b1), (W2, b2), (W3, b3) = [(w, b.reshape(1, -1)) for w, b in layers]
    c = x.shape[-1]
    w1x, w1p = W1[:c], W1[c:]
    n = posT.shape[2]
    h3 = W3.shape[1]
    spec3 = lambda s: pl.BlockSpec((1,) + s[1:], lambda b: (b, 0, 0))
    wspec = lambda s: pl.BlockSpec(s, lambda b: (0, 0))
    return pl.pallas_call(
        functools.partial(_sa_kernel, ns=ns, r2=r2, nk=min(KNBR, n),
                          x_is_pos=x_is_pos),
        grid=(B,),
        in_specs=[spec3(posT.shape), spec3(x.shape), spec3(sel.shape),
                  wspec(w1x.shape), wspec(w1p.shape), wspec(b1.shape),
                  wspec(W2.shape), wspec(b2.shape),
                  wspec(W3.shape), wspec(b3.shape)],
        out_specs=[pl.BlockSpec((1, ns, h3), lambda b: (b, 0, 0)),
                   pl.BlockSpec((1, ns, 3), lambda b: (b, 0, 0))],
        out_shape=[jax.ShapeDtypeStruct((B, ns, h3), jnp.float32),
                   jax.ShapeDtypeStruct((B, ns, 3), jnp.float32)],
        compiler_params=_PAR,
    )(posT, x, sel, w1x, w1p, b1, W2, b2, W3, b3)


def _knn3_interp(d2, xsrc, nsrc):
    jj = jax.lax.broadcasted_iota(jnp.int32, d2.shape, 1)
    num = jnp.zeros((d2.shape[0], xsrc.shape[1]), jnp.float32)
    den = jnp.zeros((d2.shape[0], 1), jnp.float32)
    for _ in range(3):
        m = jnp.min(d2, axis=1, keepdims=True)
        jmin = jnp.min(jnp.where(d2 == m, jj, nsrc), axis=1, keepdims=True)
        oh = (jj == jmin).astype(jnp.float32)
        d2 = jnp.where(jj == jmin, INF, d2)
        w = 1.0 / jnp.maximum(m, 1e-16)
        num = num + _gdot(oh, xsrc) * w
        den = den + w
    return num / den


def _mid_kernel(x2_ref, pos2_ref, pos2T_ref, x1_ref, pos1_ref,
                s31_ref, s31b, s32_ref, s32b, s33_ref, s33b,
                f31_ref, f31b, f32_ref, f32b,
                f21_ref, f21b, f22_ref, f22b,
                gmax_ref, xf2_ref):
    x2 = x2_ref[0]
    pos2 = pos2_ref[0]
    pos2T = pos2T_ref[0]
    x1 = x1_ref[0]
    pos1 = pos1_ref[0]

    g = _mlp(jnp.concatenate([x2, pos2], axis=1),
             [(s31_ref[...], s31b[...]), (s32_ref[...], s32b[...]),
              (s33_ref[...], s33b[...])])
    gmax = jnp.max(g, axis=0, keepdims=True)
    gmax_ref[0] = gmax

    d2z = (pos2[:, 0:1] ** 2 + pos2[:, 1:2] ** 2) + pos2[:, 2:3] ** 2
    w = 1.0 / jnp.maximum(d2z, 1e-16)
    y3 = (gmax * w) / w
    x3 = _mlp(jnp.concatenate([y3, x2], axis=1),
              [(f31_ref[...], f31b[...]), (f32_ref[...], f32b[...])])

    d2 = (((pos1[:, 0:1] - pos2T[0:1, :]) ** 2
           + (pos1[:, 1:2] - pos2T[1:2, :]) ** 2)
          + (pos1[:, 2:3] - pos2T[2:3, :]) ** 2)
    y2 = _knn3_interp(d2, x3, pos2T.shape[1])
    xf2_ref[0] = _mlp(jnp.concatenate([y2, x1], axis=1),
                      [(f21_ref[...], f21b[...]), (f22_ref[...], f22b[...])])


def _ape_kernel(posT_ref, loc_ref, a1_ref, a1b, a2_ref, a2b, af_ref, ap_ref):
    loc = loc_ref[...].astype(jnp.int32)
    ii = jax.lax.broadcasted_iota(jnp.int32, (B, N), 1)
    oh = ii == loc
    ax = jnp.sum(jnp.where(oh, posT_ref[:, 0, :], 0.0), axis=1, keepdims=True)
    ay = jnp.sum(jnp.where(oh, posT_ref[:, 1, :], 0.0), axis=1, keepdims=True)
    az = jnp.sum(jnp.where(oh, posT_ref[:, 2, :], 0.0), axis=1, keepdims=True)
    ap = jnp.concatenate([ax, ay, az], axis=1)
    ap_ref[...] = ap
    af_ref[:, 0, :] = _mlp(ap, [(a1_ref[...], a1b[...]), (a2_ref[...], a2b[...])])


def _big_kernel(xf2_ref, pos1T_ref, poscol_ref, af_ref,
                f11_ref, f11b, f12_ref, f12b, f13_ref, f13b,
                g1_ref, g1b, g2_ref, g2b, g3_ref, g3b,
                prob_ref):
    xf2 = xf2_ref[0]
    pos1T = pos1T_ref[0]
    poscol = poscol_ref[0]
    af = af_ref[0]

    d2 = (((poscol[:, 0:1] - pos1T[0:1, :]) ** 2
           + (poscol[:, 1:2] - pos1T[1:2, :]) ** 2)
          + (poscol[:, 2:3] - pos1T[2:3, :]) ** 2)
    y1 = _knn3_interp(d2, xf2, pos1T.shape[1])
    xf1 = _mlp(jnp.concatenate([y1, poscol], axis=1),
               [(f11_ref[...], f11b[...]), (f12_ref[...], f12b[...]),
                (f13_ref[...], f13b[...])])
    feat = jnp.concatenate([xf1, jnp.broadcast_to(af, (xf1.shape[0], af.shape[1]))],
                           axis=1)
    h = _mlp(feat, [(g1_ref[...], g1b[...]), (g2_ref[...], g2b[...]),
                    (g3_ref[...], g3b[...])])
    prob_ref[0] = jax.nn.sigmoid(h)


def _head_kernel(prob_ref, posT_ref, ap_ref, gmax_ref,
                 t1_ref, t1b, t2_ref, t2b,
                 p1_ref, p1b, p2_ref, p2b, p3_ref, p3b,
                 gr_ref, len_ref, dot_ref):
    prob = prob_ref[...]
    ii = jax.lax.broadcasted_iota(jnp.int32, (B, N), 1)
    gps = []
    for _ in range(2):
        m = jnp.max(prob, axis=1, keepdims=True)
        jmin = jnp.min(jnp.where(prob == m, ii, N), axis=1, keepdims=True)
        oh = ii == jmin
        prob = jnp.where(oh, -INF, prob)
        gx = jnp.sum(jnp.where(oh, posT_ref[:, 0, :], 0.0), axis=1, keepdims=True)
        gy = jnp.sum(jnp.where(oh, posT_ref[:, 1, :], 0.0), axis=1, keepdims=True)
        gz = jnp.sum(jnp.where(oh, posT_ref[:, 2, :], 0.0), axis=1, keepdims=True)
        gps.append(jnp.concatenate([gx, gy, gz], axis=1))
    p1, p2 = gps
    ap = ap_ref[...]
    cond = jnp.concatenate([ap, p1, p2], axis=1)
    tf = _mlp(cond, [(t1_ref[...], t1b[...]), (t2_ref[...], t2b[...])])
    trans = _mlp(jnp.concatenate([gmax_ref[...], tf], axis=1),
                 [(p1_ref[...], p1b[...]), (p2_ref[...], p2b[...]),
                  (p3_ref[...], p3b[...])])
    z = p2 - p1
    zn = jnp.sqrt(jnp.sum(z * z, axis=1, keepdims=True))
    z = z / (zn + 1e-12)
    mid = (p1 + p2) / 2.0
    xv = trans - mid
    length = jnp.sqrt(jnp.sum(xv * xv, axis=1, keepdims=True))
    xa = xv / (length + 1e-12)
    dzx = jnp.sum(z * xa, axis=1, keepdims=True)
    ya = jnp.concatenate(
        [z[:, 1:2] * xa[:, 2:3] - z[:, 2:3] * xa[:, 1:2],
         z[:, 2:3] * xa[:, 0:1] - z[:, 0:1] * xa[:, 2:3],
         z[:, 0:1] * xa[:, 1:2] - z[:, 1:2] * xa[:, 0:1]], axis=1)
    cols = []
    for i in range(3):
        cols += [xa[:, i:i + 1], ya[:, i:i + 1], z[:, i:i + 1], mid[:, i:i + 1]]
    gr_ref[...] = jnp.concatenate(cols, axis=1)
    len_ref[...] = length
    dot_ref[...] = dzx


def _full_spec(shape):
    return pl.BlockSpec(shape, lambda: tuple(0 for _ in shape))


def _mid_call(x2, pos2, pos2T, x1, pos1, params):
    (s31, s31b), (s32, s32b), (s33, s33b) = [(w, b.reshape(1, -1)) for w, b in params['sa3']]
    (f31, f31b), (f32, f32b) = [(w, b.reshape(1, -1)) for w, b in params['fp3']]
    (f21, f21b), (f22, f22b) = [(w, b.reshape(1, -1)) for w, b in params['fp2']]
    spec3 = lambda s: pl.BlockSpec((1,) + s[1:], lambda b: (b, 0, 0))
    wspec = lambda a: pl.BlockSpec(a.shape, lambda b: (0, 0))
    gmax3, xf2 = pl.pallas_call(
        _mid_kernel,
        grid=(B,),
        in_specs=[spec3(x2.shape), spec3(pos2.shape), spec3(pos2T.shape),
                  spec3(x1.shape), spec3(pos1.shape)]
                 + [wspec(a) for a in (s31, s31b, s32, s32b, s33, s33b,
                                       f31, f31b, f32, f32b,
                                       f21, f21b, f22, f22b)],
        out_specs=[pl.BlockSpec((1, 1, GFD), lambda b: (b, 0, 0)),
                   pl.BlockSpec((1, NS1, 128), lambda b: (b, 0, 0))],
        out_shape=[jax.ShapeDtypeStruct((B, 1, GFD), jnp.float32),
                   jax.ShapeDtypeStruct((B, NS1, 128), jnp.float32)],
        compiler_params=_PAR,
    )(x2, pos2, pos2T, x1, pos1,
      s31, s31b, s32, s32b, s33, s33b, f31, f31b, f32, f32b,
      f21, f21b, f22, f22b)
    return gmax3, xf2


def _ape_call(posT, aploc, params):
    (a1, a1b), (a2, a2b) = [(w, b.reshape(1, -1)) for w, b in params['ape']]
    af3, ap = pl.pallas_call(
        _ape_kernel,
        in_specs=[_full_spec(posT.shape), _full_spec((B, 1)),
                  _full_spec(a1.shape), _full_spec(a1b.shape),
                  _full_spec(a2.shape), _full_spec(a2b.shape)],
        out_specs=[_full_spec((B, 1, AFD)), _full_spec((B, 3))],
        out_shape=[jax.ShapeDtypeStruct((B, 1, AFD), jnp.float32),
                   jax.ShapeDtypeStruct((B, 3), jnp.float32)],
    )(posT, aploc, a1, a1b, a2, a2b)
    return af3, ap


def _big_call(xf2, pos1T, posr, af3, params):
    spec3 = lambda s: pl.BlockSpec((1,) + s[1:], lambda b: (b, 0, 0))
    wspec = lambda a: pl.BlockSpec(a.shape, lambda b: (0, 0))
    (f11, f11b), (f12, f12b), (f13, f13b) = [(w, b.reshape(1, -1)) for w, b in params['fp1']]
    (g1, g1b), (g2, g2b), (g3, g3b) = [(w, b.reshape(1, -1)) for w, b in params['gmlp']]
    prob3 = pl.pallas_call(
        _big_kernel,
        grid=(B,),
        in_specs=[spec3(xf2.shape), spec3(pos1T.shape), spec3(posr.shape),
                  spec3(af3.shape)]
                 + [wspec(a) for a in (f11, f11b, f12, f12b, f13, f13b,
                                       g1, g1b, g2, g2b, g3, g3b)],
        out_specs=pl.BlockSpec((1, N, 1), lambda b: (b, 0, 0)),
        out_shape=jax.ShapeDtypeStruct((B, N, 1), jnp.float32),
        compiler_params=_PAR,
    )(xf2, pos1T, posr, af3, f11, f11b, f12, f12b, f13, f13b,
      g1, g1b, g2, g2b, g3, g3b)
    return prob3


def _head_call(prob, posT, ap, gmax, params):
    (t1, t1b), (t2, t2b) = [(w, b.reshape(1, -1)) for w, b in params['tenc']]
    (q1, q1b), (q2, q2b), (q3, q3b) = [(w, b.reshape(1, -1)) for w, b in params['tpred']]
    gr12, length, dzx = pl.pallas_call(
        _head_kernel,
        in_specs=[_full_spec((B, N)), _full_spec(posT.shape), _full_spec((B, 3)),
                  _full_spec((B, GFD))]
                 + [_full_spec(a.shape) for a in (t1, t1b, t2, t2b,
                                                  q1, q1b, q2, q2b, q3, q3b)],
        out_specs=[_full_spec((B, 12)), _full_spec((B, 1)), _full_spec((B, 1))],
        out_shape=[jax.ShapeDtypeStruct((B, 12), jnp.float32),
                   jax.ShapeDtypeStruct((B, 1), jnp.float32),
                   jax.ShapeDtypeStruct((B, 1), jnp.float32)],
    )(prob, posT, ap, gmax, t1, t1b, t2, t2b, q1, q1b, q2, q2b, q3, q3b)
    return gr12, length, dzx


def kernel(pos, batch, approach_point_idx, params):
    del batch
    posr = pos.reshape(B, N, 3)
    posT = jnp.transpose(posr, (0, 2, 1))

    sel1 = _fps_call(posT, NS1)
    x1, pos1 = _sa_call(posT, posr, sel1, params['sa1'], NS1, R1 * R1, True)
    pos1T = jnp.transpose(pos1, (0, 2, 1))
    sel2 = _fps_call(pos1T, NS2)
    x2, pos2 = _sa_call(pos1T, x1, sel2, params['sa2'], NS2, R2 * R2, False)
    pos2T = jnp.transpose(pos2, (0, 2, 1))

    gmax3, xf2 = _mid_call(x2, pos2, pos2T, x1, pos1, params)

    aploc = (approach_point_idx - jnp.arange(B) * N).astype(jnp.float32).reshape(B, 1)
    af3, ap = _ape_call(posT, aploc, params)

    prob3 = _big_call(xf2, pos1T, posr, af3, params)
    prob = prob3.reshape(B, N)
    gmax = gmax3.reshape(B, GFD)

    gr12, length, dzx = _head_call(prob, posT, ap, gmax, params)
    return gr12.reshape(B, 3, 4), length.reshape(B), dzx.reshape(B)
